# baseline (speedup 1.0000x reference)
import jax
import jax.numpy as jnp
from jax import lax
from jax.experimental import pallas as pl
from jax.experimental.pallas import tpu as pltpu
from jax.experimental.pallas import tpu_sc as plsc

_SCALE = 0.95
_THRESH = 0.01

_N = 4096
_NW = 32
_ROWS_W = _N // _NW
_CH = 8
_NCHUNK = _ROWS_W // _CH
_NBUF = 3
_CHUNK = _CH * _N
_MWR = _N // 32
_MCH = _CH * _MWR
_ITERS = _CHUNK // 128


def _compute_chunk(wbuf, mbuf, cnt):
    @plsc.parallel_loop(0, _ITERS, unroll=2, carry=cnt)
    def bbody(p, acc):
        row = p >> 5
        inner = p & 31
        g = inner >> 2
        sub = inner & 3
        mwv = mbuf[pl.ds(row * 128 + g * 16, 16)]
        base = row * 4096 + g * 512 + sub * 128
        for j in range(8):
            bits = (mwv >> (sub * 8 + j)) & 1
            imp0 = bits == 0
            off = base + 16 * j
            wv = wbuf[pl.ds(off, 16)]
            sv = wv * _SCALE
            weak = jnp.abs(sv) < _THRESH
            prune = weak & imp0
            wbuf[pl.ds(off, 16)] = jnp.where(prune, 0.0, sv)
            acc = acc + plsc.all_reduce_population_count(prune)
        return acc

    return bbody


def _sc_body(w_hbm, m_hbm, out_hbm, part_hbm,
             w0, w1, w2, m0, m1, m2, cnt_buf,
             si0, si1, si2, so0, so1, so2):
    cid = lax.axis_index("c")
    sid = lax.axis_index("s")
    wid = sid * 2 + cid
    row0 = wid * _ROWS_W

    wbufs = (w0, w1, w2)
    mbufs = (m0, m1, m2)
    sis = (si0, si1, si2)
    sos = (so0, so1, so2)

    def start_in(t):
        b = t % _NBUF
        for r in range(_CH):
            row = row0 + t * _CH + r
            pltpu.async_copy(w_hbm.at[row], wbufs[b].at[pl.ds(r * _N, _N)], sis[b])
            pltpu.async_copy(m_hbm.at[row], mbufs[b].at[pl.ds(r * _MWR, _MWR)], sis[b])

    def wait_in(t):
        b = t % _NBUF
        for r in range(_CH):
            row = row0 + t * _CH + r
            pltpu.make_async_copy(w_hbm.at[row], wbufs[b].at[pl.ds(r * _N, _N)], sis[b]).wait()
            pltpu.make_async_copy(m_hbm.at[row], mbufs[b].at[pl.ds(r * _MWR, _MWR)], sis[b]).wait()

    def start_out(t):
        b = t % _NBUF
        for r in range(_CH):
            row = row0 + t * _CH + r
            pltpu.async_copy(wbufs[b].at[pl.ds(r * _N, _N)], out_hbm.at[row], sos[b])

    def wait_out(t):
        b = t % _NBUF
        for r in range(_CH):
            row = row0 + t * _CH + r
            pltpu.make_async_copy(wbufs[b].at[pl.ds(r * _N, _N)], out_hbm.at[row], sos[b]).wait()

    start_in(0)
    start_in(1)

    cnt = jnp.zeros((16,), jnp.int32)
    for t in range(_NCHUNK):
        wait_in(t)
        cnt = _compute_chunk(wbufs[t % _NBUF], mbufs[t % _NBUF], cnt)
        start_out(t)
        if t + 2 < _NCHUNK:
            if t >= 1:
                wait_out(t - 1)
            start_in(t + 2)
    wait_out(_NCHUNK - 2)
    wait_out(_NCHUNK - 1)

    cnt_buf[...] = cnt
    pltpu.sync_copy(cnt_buf, part_hbm.at[wid])


@jax.jit
def kernel(weights, importance_mask):
    shifts = jnp.arange(32, dtype=jnp.uint32)
    mbits = importance_mask.reshape(_N, 8, 32, 16).astype(jnp.uint32)
    m_words = (
        (mbits << shifts[None, None, :, None])
        .sum(axis=2, dtype=jnp.uint32)
        .reshape(_N, _MWR)
        .view(jnp.int32)
    )

    mesh = plsc.VectorSubcoreMesh(core_axis_name="c", subcore_axis_name="s")
    out, part = pl.kernel(
        _sc_body,
        out_type=[
            jax.ShapeDtypeStruct((_N, _N), jnp.float32),
            jax.ShapeDtypeStruct((_NW, 16), jnp.int32),
        ],
        mesh=mesh,
        compiler_params=pltpu.CompilerParams(needs_layout_passes=False),
        scratch_types=[
            pltpu.VMEM((_CHUNK,), jnp.float32),
            pltpu.VMEM((_CHUNK,), jnp.float32),
            pltpu.VMEM((_CHUNK,), jnp.float32),
            pltpu.VMEM((_MCH,), jnp.int32),
            pltpu.VMEM((_MCH,), jnp.int32),
            pltpu.VMEM((_MCH,), jnp.int32),
            pltpu.VMEM((16,), jnp.int32),
            pltpu.SemaphoreType.DMA,
            pltpu.SemaphoreType.DMA,
            pltpu.SemaphoreType.DMA,
            pltpu.SemaphoreType.DMA,
            pltpu.SemaphoreType.DMA,
            pltpu.SemaphoreType.DMA,
        ],
    )(weights, m_words)
    n_pruned = part[:, 0].sum().astype(jnp.int32)
    return out, n_pruned
